# Initial kernel scaffold; baseline (speedup 1.0000x reference)
import jax
import jax.numpy as jnp
from jax.experimental import pallas as pl


def kernel(x, edge_index, edge_weight, W1, b1, W2, b2):
    raise NotImplementedError("write your pallas kernel here")


# speedup vs baseline: 10.2869x; 10.2869x over previous
import functools

import jax
import jax.numpy as jnp
from jax import lax
from jax.experimental import pallas as pl
from jax.experimental.pallas import tpu as pltpu
from jax.experimental.pallas import tpu_sc as plsc

NC = 2
NS = 16
LANES = 16
CHUNK = 128

_mesh = lambda: plsc.VectorSubcoreMesh(core_axis_name="c", subcore_axis_name="s")


def _round_up(a, b):
    return (a + b - 1) // b * b


def _make_deg_kernel(np_, ep):
    edges_per_tile = ep // (NC * NS)
    n_chunks = edges_per_tile // CHUNK
    n_per_tile = np_ // NS

    def body(dst_hbm, ew_hbm, out_hbm, dst_v, ew_v, zv, deg_sh):
        c = lax.axis_index("c")
        s = lax.axis_index("s")
        base = c * (ep // NC) + s * edges_per_tile

        def zloop(i, _):
            zv[pl.ds(i * LANES, LANES)] = jnp.zeros((LANES,), jnp.float32)
            return _
        lax.fori_loop(0, n_per_tile // LANES, zloop, None)
        pltpu.sync_copy(zv, deg_sh.at[pl.ds(s * n_per_tile, n_per_tile)])
        plsc.subcore_barrier()

        def chunk_loop(k, _):
            off = base + k * CHUNK
            pltpu.sync_copy(dst_hbm.at[pl.ds(off, CHUNK)], dst_v)
            pltpu.sync_copy(ew_hbm.at[pl.ds(off, CHUNK)], ew_v)
            pltpu.sync_copy(ew_v, deg_sh.at[dst_v], add=True)
            return _
        lax.fori_loop(0, n_chunks, chunk_loop, None)
        plsc.subcore_barrier()

        pltpu.sync_copy(deg_sh.at[pl.ds(s * n_per_tile, n_per_tile)],
                        out_hbm.at[c, pl.ds(s * n_per_tile, n_per_tile)])

    return pl.kernel(
        body,
        out_type=jax.ShapeDtypeStruct((NC, np_), jnp.float32),
        mesh=_mesh(),
        scratch_types=[
            pltpu.VMEM((CHUNK,), jnp.int32),
            pltpu.VMEM((CHUNK,), jnp.float32),
            pltpu.VMEM((n_per_tile,), jnp.float32),
            pltpu.VMEM_SHARED((np_,), jnp.float32),
        ],
    )


def _make_agg_kernel(np_, ep, d):
    edges_per_tile = ep // (NC * NS)
    n_chunks = edges_per_tile // CHUNK
    n_per_tile = np_ // NS
    zrows = CHUNK

    def body(y_hbm, src_hbm, dst_hbm, ew_hbm, out_hbm,
             src_v, dst_v, ew_v, rows_v, zrow_v, acc_sh, gsem):
        c = lax.axis_index("c")
        s = lax.axis_index("s")
        base = c * (ep // NC) + s * edges_per_tile

        def zloop(i, _):
            for j in range(d // LANES):
                zrow_v[i, pl.ds(j * LANES, LANES)] = jnp.zeros((LANES,), jnp.float32)
            return _
        lax.fori_loop(0, zrows, zloop, None)
        for b in range(n_per_tile // zrows):
            pltpu.sync_copy(zrow_v, acc_sh.at[pl.ds(s * n_per_tile + b * zrows, zrows)])
        plsc.subcore_barrier()

        def chunk_loop(k, _):
            off = base + k * CHUNK
            pltpu.sync_copy(src_hbm.at[pl.ds(off, CHUNK)], src_v)
            pltpu.sync_copy(dst_hbm.at[pl.ds(off, CHUNK)], dst_v)
            pltpu.sync_copy(ew_hbm.at[pl.ds(off, CHUNK)], ew_v)
            pltpu.async_copy(y_hbm.at[src_v], rows_v, gsem).wait()

            def scale(g, _):
                wv = ew_v[pl.ds(g * LANES, LANES)]
                for i in range(LANES):
                    w = wv[i]
                    row = g * LANES + i
                    for j in range(d // LANES):
                        sl = rows_v[row, pl.ds(j * LANES, LANES)]
                        rows_v[row, pl.ds(j * LANES, LANES)] = sl * w
                return _
            lax.fori_loop(0, CHUNK // LANES, scale, None)

            pltpu.sync_copy(rows_v, acc_sh.at[dst_v], add=True)
            return _
        lax.fori_loop(0, n_chunks, chunk_loop, None)
        plsc.subcore_barrier()

        pltpu.sync_copy(acc_sh.at[pl.ds(s * n_per_tile, n_per_tile)],
                        out_hbm.at[c, pl.ds(s * n_per_tile, n_per_tile)])

    return pl.kernel(
        body,
        out_type=jax.ShapeDtypeStruct((NC, np_, d), jnp.float32),
        mesh=_mesh(),
        scratch_types=[
            pltpu.VMEM((CHUNK,), jnp.int32),
            pltpu.VMEM((CHUNK,), jnp.int32),
            pltpu.VMEM((CHUNK,), jnp.float32),
            pltpu.VMEM((CHUNK, d), jnp.float32),
            pltpu.VMEM((zrows, d), jnp.float32),
            pltpu.VMEM_SHARED((np_, d), jnp.float32),
            pltpu.SemaphoreType.DMA,
        ],
    )


def _prep_body(degp_ref, x_ref, dinv_ref, y_ref):
    deg = degp_ref[0] + degp_ref[1] + 1.0
    dv = jnp.where(deg > 0, lax.rsqrt(deg), 0.0)
    dinv_ref[...] = dv
    y_ref[...] = dv * x_ref[...]


def _mid_body(aggp_ref, x_ref, dinv_ref, w1_ref, b1_ref, w2_ref, t_ref, y2_ref):
    dv = dinv_ref[...]
    ax = dv * (aggp_ref[0] + aggp_ref[1]) + (dv * dv) * x_ref[...]
    h = jnp.maximum(
        jnp.dot(ax, w1_ref[...], preferred_element_type=jnp.float32) + b1_ref[...],
        0.0)
    t = jnp.dot(h, w2_ref[...], preferred_element_type=jnp.float32)
    t_ref[...] = t
    y2_ref[...] = dv * t


def _final_body(aggp_ref, t_ref, dinv_ref, b2_ref, out_ref):
    dv = dinv_ref[...]
    out_ref[...] = (dv * (aggp_ref[0] + aggp_ref[1])
                    + (dv * dv) * t_ref[...] + b2_ref[...])


def kernel(x, edge_index, edge_weight, W1, b1, W2, b2):
    n, d_in = x.shape
    d_hid = W1.shape[1]
    d_out = W2.shape[1]
    e = edge_weight.shape[0]

    np_ = _round_up(n, NS * CHUNK)
    ep = _round_up(e, NC * NS * CHUNK)

    src = edge_index[0].astype(jnp.int32)
    dst = edge_index[1].astype(jnp.int32)
    ew = edge_weight.astype(jnp.float32)
    pad_e = ep - e
    src = jnp.concatenate([src, jnp.zeros((pad_e,), jnp.int32)])
    dst = jnp.concatenate([dst, jnp.zeros((pad_e,), jnp.int32)])
    ew = jnp.concatenate([ew, jnp.zeros((pad_e,), jnp.float32)])
    xp = jnp.concatenate([x, jnp.zeros((np_ - n, d_in), x.dtype)])

    degp = _make_deg_kernel(np_, ep)(dst, ew)

    r = 512
    grid = (np_ // r,)
    dinv, y = pl.pallas_call(
        _prep_body,
        grid=grid,
        in_specs=[
            pl.BlockSpec((NC, r, 1), lambda i: (0, i, 0)),
            pl.BlockSpec((r, d_in), lambda i: (i, 0)),
        ],
        out_specs=[
            pl.BlockSpec((r, 1), lambda i: (i, 0)),
            pl.BlockSpec((r, d_in), lambda i: (i, 0)),
        ],
        out_shape=[
            jax.ShapeDtypeStruct((np_, 1), jnp.float32),
            jax.ShapeDtypeStruct((np_, d_in), jnp.float32),
        ],
    )(degp.reshape(NC, np_, 1), xp)

    agg1 = _make_agg_kernel(np_, ep, d_in)(y, src, dst, ew)

    t, y2 = pl.pallas_call(
        _mid_body,
        grid=grid,
        in_specs=[
            pl.BlockSpec((NC, r, d_in), lambda i: (0, i, 0)),
            pl.BlockSpec((r, d_in), lambda i: (i, 0)),
            pl.BlockSpec((r, 1), lambda i: (i, 0)),
            pl.BlockSpec((d_in, d_hid), lambda i: (0, 0)),
            pl.BlockSpec((1, d_hid), lambda i: (0, 0)),
            pl.BlockSpec((d_hid, d_out), lambda i: (0, 0)),
        ],
        out_specs=[
            pl.BlockSpec((r, d_out), lambda i: (i, 0)),
            pl.BlockSpec((r, d_out), lambda i: (i, 0)),
        ],
        out_shape=[
            jax.ShapeDtypeStruct((np_, d_out), jnp.float32),
            jax.ShapeDtypeStruct((np_, d_out), jnp.float32),
        ],
    )(agg1, xp, dinv, W1, b1.reshape(1, d_hid), W2)

    agg2 = _make_agg_kernel(np_, ep, d_out)(y2, src, dst, ew)

    out = pl.pallas_call(
        _final_body,
        grid=grid,
        in_specs=[
            pl.BlockSpec((NC, r, d_out), lambda i: (0, i, 0)),
            pl.BlockSpec((r, d_out), lambda i: (i, 0)),
            pl.BlockSpec((r, 1), lambda i: (i, 0)),
            pl.BlockSpec((1, d_out), lambda i: (0, 0)),
        ],
        out_specs=pl.BlockSpec((r, d_out), lambda i: (i, 0)),
        out_shape=jax.ShapeDtypeStruct((np_, d_out), jnp.float32),
    )(agg2, t, dinv, b2.reshape(1, d_out))

    return out[:n]
